# baseline (device time: 72621 ns/iter reference)
import jax
import jax.numpy as jnp
from jax import lax
from jax.experimental import pallas as pl
from jax.experimental.pallas import tpu as pltpu

N_DEV = 16
NZ = 4
NP = 4


def kernel(x, w_mat, scale_x, scale_w):
    m_per, k = x.shape
    _, n = w_mat.shape
    n_per = n // N_DEV
    n_plane = NP * n_per
    m_tot = N_DEV * m_per

    def body(x_ref, w_hbm, sx_ref, sw_ref, out_ref,
             xg_ref, stage_ref, w8_ref, yb_ref, yr_ref,
             w_sems, zsend, zrecv, ysend, yrecv):
        me = lax.axis_index("i")
        z_me = me // NP
        p_me = me % NP
        base = NP * z_me
        scale = sx_ref[0] * sw_ref[0]

        xg_ref[pl.ds(z_me * m_per, m_per), :] = (
            x_ref[...].astype(jnp.float8_e4m3fn))

        def z_rdma(slot, dirn, target):
            return pltpu.make_async_remote_copy(
                src_ref=xg_ref.at[pl.ds(slot * m_per, m_per), :],
                dst_ref=xg_ref.at[pl.ds(slot * m_per, m_per), :],
                send_sem=zsend.at[dirn, slot],
                recv_sem=zrecv.at[slot],
                device_id=(target,),
                device_id_type=pl.DeviceIdType.MESH,
            )

        def y_rdma(zp, q):
            return pltpu.make_async_remote_copy(
                src_ref=yb_ref.at[pl.ds(zp * m_per, m_per),
                                  pl.ds(q * n_per, n_per)],
                dst_ref=yr_ref.at[pl.ds((NP * zp + p_me) * m_per, m_per), :],
                send_sem=ysend.at[q, zp],
                recv_sem=yrecv.at[p_me, zp],
                device_id=(base + q,),
                device_id_type=pl.DeviceIdType.MESH,
            )

        @pl.when(z_me <= NZ - 2)
        def _():
            z_rdma(z_me, 0, me + NP).start()

        @pl.when(z_me >= 1)
        def _():
            z_rdma(z_me, 1, me - NP).start()

        wcps = [
            pltpu.make_async_copy(
                w_hbm.at[:, pl.ds(z_me * n_plane + c * n_per, n_per)],
                stage_ref.at[c % 2],
                w_sems.at[c % 2],
            )
            for c in range(NP)
        ]
        wcps[0].start()
        wcps[1].start()
        for c in range(NP):
            wcps[c].wait()
            w8_ref[:, pl.ds(c * n_per, n_per)] = (
                stage_ref[c % 2].astype(jnp.float8_e5m2))
            if c + 2 < NP:
                wcps[c + 2].start()

        def compute_scatter(zp):
            xa = xg_ref[pl.ds(zp * m_per, m_per), :]
            acc = lax.dot_general(
                xa, w8_ref[...], (((1,), (0,)), ((), ())),
                preferred_element_type=jnp.float32)
            y = acc * scale
            yv = y * jax.nn.sigmoid(jnp.clip(y, -60.0, 60.0))
            yb_ref[pl.ds(zp * m_per, m_per), :] = yv.astype(jnp.bfloat16)
            for dq in (1, 2, 3):
                y_rdma(zp, (p_me + dq) % NP).start()
            out_ref[pl.ds((NP * zp + p_me) * m_per, m_per), :] = (
                yb_ref[pl.ds(zp * m_per, m_per),
                       pl.ds(p_me * n_per, n_per)].astype(jnp.float32)
            )

        compute_scatter(z_me)

        for s in range(NZ - 1):
            up_ok = z_me >= s + 1
            dn_ok = z_me <= NZ - 2 - s

            @pl.when(up_ok)
            def _():
                z_rdma(z_me - 1 - s, 0, me + NP).wait_recv()

            @pl.when(up_ok & (z_me <= NZ - 2))
            def _():
                z_rdma(z_me - 1 - s, 0, me + NP).start()

            @pl.when(dn_ok)
            def _():
                z_rdma(z_me + 1 + s, 1, me - NP).wait_recv()

            @pl.when(dn_ok & (z_me >= 1))
            def _():
                z_rdma(z_me + 1 + s, 1, me - NP).start()

            @pl.when(up_ok)
            def _():
                compute_scatter(z_me - 1 - s)

            @pl.when(dn_ok)
            def _():
                compute_scatter(z_me + 1 + s)

        for dq in (1, 2, 3):
            q = (p_me + dq) % NP
            for zp in range(NZ):
                recv = pltpu.make_async_remote_copy(
                    src_ref=yb_ref.at[pl.ds(zp * m_per, m_per),
                                      pl.ds(q * n_per, n_per)],
                    dst_ref=yr_ref.at[pl.ds((NP * zp + q) * m_per, m_per), :],
                    send_sem=ysend.at[q, zp],
                    recv_sem=yrecv.at[q, zp],
                    device_id=(base + q,),
                    device_id_type=pl.DeviceIdType.MESH,
                )
                recv.wait_recv()
                out_ref[pl.ds((NP * zp + q) * m_per, m_per), :] = (
                    yr_ref[pl.ds((NP * zp + q) * m_per, m_per), :]
                    .astype(jnp.float32)
                )

        @pl.when(z_me <= NZ - 2)
        def _():
            z_rdma(z_me, 0, me + NP).wait_send()

        @pl.when(z_me >= 1)
        def _():
            z_rdma(z_me, 1, me - NP).wait_send()

        for s in range(NZ - 1):
            @pl.when((z_me >= s + 1) & (z_me <= NZ - 2))
            def _():
                z_rdma(z_me - 1 - s, 0, me + NP).wait_send()

            @pl.when((z_me <= NZ - 2 - s) & (z_me >= 1))
            def _():
                z_rdma(z_me + 1 + s, 1, me - NP).wait_send()

        for dq in (1, 2, 3):
            for zp in range(NZ):
                y_rdma(zp, (p_me + dq) % NP).wait_send()

    return pl.pallas_call(
        body,
        out_shape=jax.ShapeDtypeStruct((m_tot, n_per), jnp.float32),
        in_specs=[
            pl.BlockSpec(memory_space=pltpu.VMEM),
            pl.BlockSpec(memory_space=pl.ANY),
            pl.BlockSpec(memory_space=pltpu.SMEM),
            pl.BlockSpec(memory_space=pltpu.SMEM),
        ],
        out_specs=pl.BlockSpec(memory_space=pltpu.VMEM),
        scratch_shapes=[
            pltpu.VMEM((NZ * m_per, k), jnp.float8_e4m3fn),
            pltpu.VMEM((2, k, n_per), jnp.float32),
            pltpu.VMEM((k, n_plane), jnp.float8_e5m2),
            pltpu.VMEM((NZ * m_per, n_plane), jnp.bfloat16),
            pltpu.VMEM((m_tot, n_per), jnp.bfloat16),
            pltpu.SemaphoreType.DMA((2,)),
            pltpu.SemaphoreType.DMA((2, NZ)),
            pltpu.SemaphoreType.DMA((NZ,)),
            pltpu.SemaphoreType.DMA((NP, NZ)),
            pltpu.SemaphoreType.DMA((NP, NZ)),
        ],
        compiler_params=pltpu.CompilerParams(
            vmem_limit_bytes=56 * 1024 * 1024),
    )(x, w_mat, scale_x, scale_w)


# device time: 71414 ns/iter; 1.0169x vs baseline; 1.0169x over previous
import jax
import jax.numpy as jnp
from jax import lax
from jax.experimental import pallas as pl
from jax.experimental.pallas import tpu as pltpu

N_DEV = 16
NZ = 4
NP = 4


def kernel(x, w_mat, scale_x, scale_w):
    m_per, k = x.shape
    _, n = w_mat.shape
    n_per = n // N_DEV
    n_plane = NP * n_per
    m_tot = N_DEV * m_per

    def body(x_ref, w_hbm, sx_ref, sw_ref, out_ref,
             xg_ref, stage_ref, w8_ref, yb_ref, yr_ref,
             w_sems, zsend, zrecv, ysend, yrecv):
        me = lax.axis_index("i")
        z_me = me // NP
        p_me = me % NP
        base = NP * z_me
        scale = sx_ref[0] * sw_ref[0]

        xg_ref[pl.ds(z_me * m_per, m_per), :] = (
            x_ref[...].astype(jnp.float8_e4m3fn))

        def z_rdma(slot, dirn, target):
            return pltpu.make_async_remote_copy(
                src_ref=xg_ref.at[pl.ds(slot * m_per, m_per), :],
                dst_ref=xg_ref.at[pl.ds(slot * m_per, m_per), :],
                send_sem=zsend.at[dirn, slot],
                recv_sem=zrecv.at[slot],
                device_id=(target,),
                device_id_type=pl.DeviceIdType.MESH,
            )

        def y_rdma(zp, q):
            return pltpu.make_async_remote_copy(
                src_ref=yb_ref.at[pl.ds(zp * m_per, m_per),
                                  pl.ds(q * n_per, n_per)],
                dst_ref=yr_ref.at[pl.ds((NP * zp + p_me) * m_per, m_per), :],
                send_sem=ysend.at[q, zp],
                recv_sem=yrecv.at[p_me, zp],
                device_id=(base + q,),
                device_id_type=pl.DeviceIdType.MESH,
            )

        @pl.when(z_me <= NZ - 2)
        def _():
            z_rdma(z_me, 0, me + NP).start()

        @pl.when(z_me >= 1)
        def _():
            z_rdma(z_me, 1, me - NP).start()

        def epilogue(acc):
            y = acc * scale
            return y * jax.nn.sigmoid(jnp.clip(y, -60.0, 60.0))

        wcps = [
            pltpu.make_async_copy(
                w_hbm.at[:, pl.ds(z_me * n_plane + c * n_per, n_per)],
                stage_ref.at[c % 2],
                w_sems.at[c % 2],
            )
            for c in range(NP)
        ]
        wcps[0].start()
        wcps[1].start()
        for c in range(NP):
            wcps[c].wait()
            w8_ref[:, pl.ds(c * n_per, n_per)] = (
                stage_ref[c % 2].astype(jnp.float8_e4m3fn))
            if c + 2 < NP:
                wcps[c + 2].start()
            xa = xg_ref[pl.ds(z_me * m_per, m_per), :]
            acc = lax.dot_general(
                xa, w8_ref[:, pl.ds(c * n_per, n_per)],
                (((1,), (0,)), ((), ())),
                preferred_element_type=jnp.float32)
            yv = epilogue(acc)
            yb_ref[pl.ds(z_me * m_per, m_per), pl.ds(c * n_per, n_per)] = (
                yv.astype(jnp.bfloat16))

            @pl.when(c == p_me)
            def _():
                out_ref[pl.ds((NP * z_me + p_me) * m_per, m_per), :] = yv

            @pl.when(c != p_me)
            def _():
                y_rdma(z_me, c).start()

        def compute_scatter(zp):
            xa = xg_ref[pl.ds(zp * m_per, m_per), :]
            acc = lax.dot_general(
                xa, w8_ref[...], (((1,), (0,)), ((), ())),
                preferred_element_type=jnp.float32)
            yv = epilogue(acc)
            yb_ref[pl.ds(zp * m_per, m_per), :] = yv.astype(jnp.bfloat16)
            for dq in (2, 1, 3):
                y_rdma(zp, (p_me + dq) % NP).start()
            out_ref[pl.ds((NP * zp + p_me) * m_per, m_per), :] = (
                yb_ref[pl.ds(zp * m_per, m_per),
                       pl.ds(p_me * n_per, n_per)].astype(jnp.float32)
            )

        for s in range(NZ - 1):
            up_ok = z_me >= s + 1
            dn_ok = z_me <= NZ - 2 - s

            @pl.when(up_ok)
            def _():
                z_rdma(z_me - 1 - s, 0, me + NP).wait_recv()

            @pl.when(up_ok & (z_me <= NZ - 2))
            def _():
                z_rdma(z_me - 1 - s, 0, me + NP).start()

            @pl.when(dn_ok)
            def _():
                z_rdma(z_me + 1 + s, 1, me - NP).wait_recv()

            @pl.when(dn_ok & (z_me >= 1))
            def _():
                z_rdma(z_me + 1 + s, 1, me - NP).start()

            @pl.when(up_ok)
            def _():
                compute_scatter(z_me - 1 - s)

            @pl.when(dn_ok)
            def _():
                compute_scatter(z_me + 1 + s)

        for dq in (1, 2, 3):
            q = (p_me + dq) % NP
            for zp in range(NZ):
                recv = pltpu.make_async_remote_copy(
                    src_ref=yb_ref.at[pl.ds(zp * m_per, m_per),
                                      pl.ds(q * n_per, n_per)],
                    dst_ref=yr_ref.at[pl.ds((NP * zp + q) * m_per, m_per), :],
                    send_sem=ysend.at[q, zp],
                    recv_sem=yrecv.at[q, zp],
                    device_id=(base + q,),
                    device_id_type=pl.DeviceIdType.MESH,
                )
                recv.wait_recv()
                out_ref[pl.ds((NP * zp + q) * m_per, m_per), :] = (
                    yr_ref[pl.ds((NP * zp + q) * m_per, m_per), :]
                    .astype(jnp.float32)
                )

        @pl.when(z_me <= NZ - 2)
        def _():
            z_rdma(z_me, 0, me + NP).wait_send()

        @pl.when(z_me >= 1)
        def _():
            z_rdma(z_me, 1, me - NP).wait_send()

        for s in range(NZ - 1):
            @pl.when((z_me >= s + 1) & (z_me <= NZ - 2))
            def _():
                z_rdma(z_me - 1 - s, 0, me + NP).wait_send()

            @pl.when((z_me <= NZ - 2 - s) & (z_me >= 1))
            def _():
                z_rdma(z_me + 1 + s, 1, me - NP).wait_send()

        for dq in (1, 2, 3):
            for zp in range(NZ):
                y_rdma(zp, (p_me + dq) % NP).wait_send()

    return pl.pallas_call(
        body,
        out_shape=jax.ShapeDtypeStruct((m_tot, n_per), jnp.float32),
        in_specs=[
            pl.BlockSpec(memory_space=pltpu.VMEM),
            pl.BlockSpec(memory_space=pl.ANY),
            pl.BlockSpec(memory_space=pltpu.SMEM),
            pl.BlockSpec(memory_space=pltpu.SMEM),
        ],
        out_specs=pl.BlockSpec(memory_space=pltpu.VMEM),
        scratch_shapes=[
            pltpu.VMEM((NZ * m_per, k), jnp.float8_e4m3fn),
            pltpu.VMEM((2, k, n_per), jnp.float32),
            pltpu.VMEM((k, n_plane), jnp.float8_e4m3fn),
            pltpu.VMEM((NZ * m_per, n_plane), jnp.bfloat16),
            pltpu.VMEM((m_tot, n_per), jnp.bfloat16),
            pltpu.SemaphoreType.DMA((2,)),
            pltpu.SemaphoreType.DMA((2, NZ)),
            pltpu.SemaphoreType.DMA((NZ,)),
            pltpu.SemaphoreType.DMA((NP, NZ)),
            pltpu.SemaphoreType.DMA((NP, NZ)),
        ],
        compiler_params=pltpu.CompilerParams(
            vmem_limit_bytes=56 * 1024 * 1024),
    )(x, w_mat, scale_x, scale_w)
